# Initial kernel scaffold; baseline (speedup 1.0000x reference)
import jax
import jax.numpy as jnp
from jax.experimental import pallas as pl


def kernel(feat0, feat1, feat2, mask0, mask1, mask2):
    raise NotImplementedError("write your pallas kernel here")


# speedup vs baseline: 165.3813x; 165.3813x over previous
import jax
import jax.numpy as jnp
from jax.experimental import pallas as pl

_BLK = 16
_H = 128
_W = 128
_C = 96
_NT = _W // _BLK
_NU = 8
_NSTEP = 2 * (_H // _BLK)


def _loss_kernel(f0, f1, f2, m0, m1, m2, out_ref):
    i = pl.program_id(0)

    tw = jax.lax.broadcasted_iota(jnp.int32, (_W, _NT), 0) // _BLK
    tt = jax.lax.broadcasted_iota(jnp.int32, (_W, _NT), 1)
    tsel = (tw == tt).astype(jnp.float32)

    protos = []
    counts = []
    for f_ref, m_ref in ((f0, m0), (f1, m1), (f2, m2)):
        f = f_ref[0]
        m = m_ref[0, 0]
        sumsq = jnp.sum(f * f, axis=0)
        inv = 1.0 / jnp.maximum(jnp.sqrt(sumsq), 1e-12)
        fn = f * inv[None]
        gs = []
        cs = []
        for u in range(_NU):
            mu = (m == u).astype(jnp.float32)
            gs.append(jnp.sum(fn * mu[None], axis=1))
            cs.append(jnp.sum(mu, axis=0, keepdims=True))
        g = jnp.concatenate(gs, axis=0)
        cw = jnp.concatenate(cs, axis=0)
        ps = jax.lax.dot_general(
            g, tsel, (((1,), (0,)), ((), ())),
            preferred_element_type=jnp.float32)
        cnt = jax.lax.dot_general(
            cw, tsel, (((1,), (0,)), ((), ())),
            preferred_element_type=jnp.float32)
        cden = jnp.maximum(cnt, 1.0)
        cbig = jnp.repeat(cden, _C, axis=0)
        protos.append(ps / cbig)
        counts.append(cnt)

    present = (counts[0] > 0) & (counts[1] > 0) & (counts[2] > 0)
    d = jnp.abs(protos[0] - 2.0 * protos[1] + protos[2])

    su = jax.lax.broadcasted_iota(jnp.int32, (_NU, _NU * _C), 0)
    sj = jax.lax.broadcasted_iota(jnp.int32, (_NU, _NU * _C), 1) // _C
    ssel = (su == sj).astype(jnp.float32)
    tsum = jax.lax.dot_general(
        ssel, d, (((1,), (0,)), ((), ())),
        preferred_element_type=jnp.float32) * (1.0 / _C)

    nclass = jnp.sum(present.astype(jnp.float32), axis=0, keepdims=True)
    lsum = jnp.sum(jnp.where(present, tsum, 0.0), axis=0, keepdims=True)
    loss_t = lsum / jnp.maximum(nclass, 1.0)
    has = (nclass > 0).astype(jnp.float32)
    strip_loss = jnp.sum(loss_t * has)
    strip_cnt = jnp.sum(has)

    r = jax.lax.broadcasted_iota(jnp.int32, (8, 128), 0)
    c = jax.lax.broadcasted_iota(jnp.int32, (8, 128), 1)
    vec = (jnp.where((r == 0) & (c == 0), strip_loss, 0.0)
           + jnp.where((r == 0) & (c == 1), strip_cnt, 0.0))

    @pl.when(i == 0)
    def _():
        out_ref[...] = jnp.zeros_like(out_ref)

    out_ref[...] += vec

    @pl.when(i == _NSTEP - 1)
    def _():
        acc = out_ref[...]
        tl = jnp.sum(jnp.where((r == 0) & (c == 0), acc, 0.0))
        tc = jnp.sum(jnp.where((r == 0) & (c == 1), acc, 0.0))
        final = jnp.where(tc > 0, tl / jnp.maximum(tc, 1.0), tl)
        out_ref[...] = jnp.where((r == 0) & (c == 0), final, 0.0)


def kernel(feat0, feat1, feat2, mask0, mask1, mask2):
    masks = [m.astype(jnp.int32) for m in (mask0, mask1, mask2)]
    fspec = pl.BlockSpec((1, _C, _BLK, _W), lambda i: (i // 8, 0, i % 8, 0))
    mspec = pl.BlockSpec((1, 1, _BLK, _W), lambda i: (i // 8, 0, i % 8, 0))
    out = pl.pallas_call(
        _loss_kernel,
        grid=(_NSTEP,),
        in_specs=[fspec, fspec, fspec, mspec, mspec, mspec],
        out_specs=pl.BlockSpec((8, 128), lambda i: (0, 0)),
        out_shape=jax.ShapeDtypeStruct((8, 128), jnp.float32),
    )(feat0, feat1, feat2, *masks)
    return out[0, 0]


# speedup vs baseline: 329.1245x; 1.9901x over previous
import jax
import jax.numpy as jnp
from jax.experimental import pallas as pl

_BLK = 16
_H = 128
_W = 128
_C = 96
_NT = _W // _BLK
_NU = 8
_NK = _NT * _NU
_P = _BLK * _W
_NSTEP = 2 * (_H // _BLK)


def _loss_kernel(f0, f1, f2, m0, m1, m2, out_ref):
    i = pl.program_id(0)

    kk = jax.lax.broadcasted_iota(jnp.int32, (_NK, _BLK, _W), 0)
    wi = jax.lax.broadcasted_iota(jnp.int32, (_NK, _BLK, _W), 2) // _BLK
    cls_of_k = kk % _NU
    tile_of_k = kk // _NU

    protos = []
    counts = []
    ones_row = jnp.ones((1, _P), jnp.float32)
    for f_ref, m_ref in ((f0, m0), (f1, m1), (f2, m2)):
        f = f_ref[0]
        m = m_ref[0, 0]
        sumsq = jnp.sum(f * f, axis=0)
        inv = 1.0 / jnp.maximum(jnp.sqrt(sumsq), 1e-12)
        fn = (f * inv[None]).reshape(_C, _P)
        oh = ((m[None] == cls_of_k) & (wi == tile_of_k)).astype(jnp.float32)
        oh = oh.reshape(_NK, _P)
        ps = jax.lax.dot_general(
            fn, oh, (((1,), (1,)), ((), ())),
            preferred_element_type=jnp.float32)
        cnt = jax.lax.dot_general(
            ones_row, oh, (((1,), (1,)), ((), ())),
            preferred_element_type=jnp.float32)
        protos.append(ps / jnp.maximum(cnt, 1.0))
        counts.append(cnt)

    present = (counts[0] > 0) & (counts[1] > 0) & (counts[2] > 0)
    d = jnp.abs(protos[0] - 2.0 * protos[1] + protos[2])
    t = jnp.sum(d, axis=0, keepdims=True) * (1.0 / _C)

    sk = jax.lax.broadcasted_iota(jnp.int32, (_NK, _NT), 0) // _NU
    st = jax.lax.broadcasted_iota(jnp.int32, (_NK, _NT), 1)
    ksel = (sk == st).astype(jnp.float32)

    pres_f = present.astype(jnp.float32)
    nclass = jax.lax.dot_general(
        pres_f, ksel, (((1,), (0,)), ((), ())),
        preferred_element_type=jnp.float32)
    lsum = jax.lax.dot_general(
        jnp.where(present, t, 0.0), ksel, (((1,), (0,)), ((), ())),
        preferred_element_type=jnp.float32)
    loss_t = lsum / jnp.maximum(nclass, 1.0)
    has = (nclass > 0).astype(jnp.float32)
    strip_loss = jnp.sum(loss_t * has)
    strip_cnt = jnp.sum(has)

    r = jax.lax.broadcasted_iota(jnp.int32, (8, 128), 0)
    c = jax.lax.broadcasted_iota(jnp.int32, (8, 128), 1)
    vec = (jnp.where((r == 0) & (c == 0), strip_loss, 0.0)
           + jnp.where((r == 0) & (c == 1), strip_cnt, 0.0))

    @pl.when(i == 0)
    def _():
        out_ref[...] = jnp.zeros_like(out_ref)

    out_ref[...] += vec

    @pl.when(i == _NSTEP - 1)
    def _():
        acc = out_ref[...]
        tl = jnp.sum(jnp.where((r == 0) & (c == 0), acc, 0.0))
        tc = jnp.sum(jnp.where((r == 0) & (c == 1), acc, 0.0))
        final = jnp.where(tc > 0, tl / jnp.maximum(tc, 1.0), tl)
        out_ref[...] = jnp.where((r == 0) & (c == 0), final, 0.0)


def kernel(feat0, feat1, feat2, mask0, mask1, mask2):
    masks = [m.astype(jnp.int32) for m in (mask0, mask1, mask2)]
    fspec = pl.BlockSpec((1, _C, _BLK, _W), lambda i: (i // 8, 0, i % 8, 0))
    mspec = pl.BlockSpec((1, 1, _BLK, _W), lambda i: (i // 8, 0, i % 8, 0))
    out = pl.pallas_call(
        _loss_kernel,
        grid=(_NSTEP,),
        in_specs=[fspec, fspec, fspec, mspec, mspec, mspec],
        out_specs=pl.BlockSpec((8, 128), lambda i: (0, 0)),
        out_shape=jax.ShapeDtypeStruct((8, 128), jnp.float32),
    )(feat0, feat1, feat2, *masks)
    return out[0, 0]


# speedup vs baseline: 368.8330x; 1.1206x over previous
import jax
import jax.numpy as jnp
from jax.experimental import pallas as pl

_BLK = 16
_H = 128
_W = 128
_C = 96
_NT = _W // _BLK
_NU = 8
_NK = _NT * _NU
_P = _BLK * _W
_NSTEP = 2 * (_H // _BLK)


def _loss_kernel(f0, f1, f2, m0, m1, m2, out_ref):
    i = pl.program_id(0)

    lane = jax.lax.broadcasted_iota(jnp.int32, (1, _P), 1)
    tile8 = (lane % _W) // _BLK * _NU
    kk = jax.lax.broadcasted_iota(jnp.int32, (_NK, _P), 0)
    ones_row = jnp.ones((1, _P), jnp.float32)
    ones_c = jnp.ones((1, _C), jnp.float32)

    protos = []
    counts = []
    for f_ref, m_ref in ((f0, m0), (f1, m1), (f2, m2)):
        fp = f_ref[0].reshape(_C, _P)
        m2 = m_ref[0, 0]
        fpsq = fp * fp
        sumsq = jax.lax.dot_general(
            ones_c, fpsq, (((1,), (0,)), ((), ())),
            preferred_element_type=jnp.float32)
        inv = 1.0 / jnp.maximum(jnp.sqrt(sumsq), 1e-12)
        key = m2 + tile8
        oh = (key == kk).astype(jnp.float32)
        cnt = jax.lax.dot_general(
            ones_row, oh, (((1,), (1,)), ((), ())),
            preferred_element_type=jnp.float32)
        oh_w = oh * inv
        ps = jax.lax.dot_general(
            fp, oh_w, (((1,), (1,)), ((), ())),
            preferred_element_type=jnp.float32)
        protos.append(ps / jnp.maximum(cnt, 1.0))
        counts.append(cnt)

    present = (counts[0] > 0) & (counts[1] > 0) & (counts[2] > 0)
    d = jnp.abs(protos[0] - 2.0 * protos[1] + protos[2])
    t = jnp.sum(d, axis=0, keepdims=True) * (1.0 / _C)

    sk = jax.lax.broadcasted_iota(jnp.int32, (_NK, _NT), 0) // _NU
    st = jax.lax.broadcasted_iota(jnp.int32, (_NK, _NT), 1)
    ksel = (sk == st).astype(jnp.float32)

    pres_f = present.astype(jnp.float32)
    nclass = jax.lax.dot_general(
        pres_f, ksel, (((1,), (0,)), ((), ())),
        preferred_element_type=jnp.float32)
    lsum = jax.lax.dot_general(
        jnp.where(present, t, 0.0), ksel, (((1,), (0,)), ((), ())),
        preferred_element_type=jnp.float32)
    loss_t = lsum / jnp.maximum(nclass, 1.0)
    has = (nclass > 0).astype(jnp.float32)
    strip_loss = jnp.sum(loss_t * has)
    strip_cnt = jnp.sum(has)

    r = jax.lax.broadcasted_iota(jnp.int32, (8, 128), 0)
    c = jax.lax.broadcasted_iota(jnp.int32, (8, 128), 1)
    vec = (jnp.where((r == 0) & (c == 0), strip_loss, 0.0)
           + jnp.where((r == 0) & (c == 1), strip_cnt, 0.0))

    @pl.when(i == 0)
    def _():
        out_ref[...] = jnp.zeros_like(out_ref)

    out_ref[...] += vec

    @pl.when(i == _NSTEP - 1)
    def _():
        acc = out_ref[...]
        tl = jnp.sum(jnp.where((r == 0) & (c == 0), acc, 0.0))
        tc = jnp.sum(jnp.where((r == 0) & (c == 1), acc, 0.0))
        final = jnp.where(tc > 0, tl / jnp.maximum(tc, 1.0), tl)
        out_ref[...] = jnp.where((r == 0) & (c == 0), final, 0.0)


def kernel(feat0, feat1, feat2, mask0, mask1, mask2):
    masks = [m.astype(jnp.int32).reshape(2, _H // _BLK, 1, _P)
             for m in (mask0, mask1, mask2)]
    fspec = pl.BlockSpec((1, _C, _BLK, _W), lambda i: (i // 8, 0, i % 8, 0))
    mspec = pl.BlockSpec((1, 1, 1, _P), lambda i: (i // 8, i % 8, 0, 0))
    out = pl.pallas_call(
        _loss_kernel,
        grid=(_NSTEP,),
        in_specs=[fspec, fspec, fspec, mspec, mspec, mspec],
        out_specs=pl.BlockSpec((8, 128), lambda i: (0, 0)),
        out_shape=jax.ShapeDtypeStruct((8, 128), jnp.float32),
    )(feat0, feat1, feat2, *masks)
    return out[0, 0]
